# baseline (speedup 1.0000x reference)
import functools

import jax
import jax.numpy as jnp
from jax import lax
from jax.experimental import pallas as pl
from jax.experimental.pallas import tpu as pltpu
from jax.experimental.pallas import tpu_sc as plsc

P = 50
PPAD = 56
PA = 48
D = 512
B = 4096
NW = 32
B_PER_W = B // NW
NBUF = 4
NITER = B_PER_W // NBUF
BB = 256


def _build_table_tc(inp_ref, emb_ref, pos_ref, t_ref, idx_ref):
    emb = emb_ref[...]
    pos = pos_ref[...]
    t_ref[...] = pos[:, None, :] + emb[None, :, :]
    inp_pad = jnp.concatenate(
        [inp_ref[...], jnp.zeros((B, PPAD - P), jnp.int32)], axis=1)
    j = lax.broadcasted_iota(jnp.int32, (B, PPAD), 1)
    idx_ref[...] = jnp.where(j < P, inp_pad + P * j, 0)


def _sc_gather(t_hbm, idx_hbm, out_hbm, idx_v, bufs, gsems, ssems):
    wid = lax.axis_index("s") * 2 + lax.axis_index("c")
    b0 = wid * B_PER_W
    pltpu.sync_copy(idx_hbm.at[pl.ds(b0 * PPAD, B_PER_W * PPAD)], idx_v)

    def gather(bl, slot):
        off = pl.multiple_of(bl * PPAD, PPAD)
        return pltpu.make_async_copy(
            t_hbm.at[idx_v.at[pl.ds(off, PPAD)]], bufs[slot], gsems[slot])

    def store(bl, slot):
        return pltpu.make_async_copy(
            bufs[slot], out_hbm.at[b0 + bl, pl.ds(0, PPAD)], ssems[slot])

    gather(0, 0).start()

    def body(g, carry):
        c0 = NBUF * g
        for j in range(NBUF):
            c = c0 + j
            nslot = (j + 1) % NBUF
            if j < NBUF - 1:
                @pl.when(g > 0)
                def _():
                    store(c + 1 - NBUF, nslot).wait()
                gather(c + 1, nslot).start()
            else:
                @pl.when(g < NITER - 1)
                def _():
                    store(c + 1 - NBUF, nslot).wait()
                    gather(c + 1, nslot).start()
            gather(c, j).wait()
            store(c, j).start()
        return carry

    lax.fori_loop(0, NITER, body, 0)
    for j in range(NBUF):
        store(B_PER_W - NBUF + j, j).wait()


def _fix_tail_tc(inp_ref, t_ref, prev_ref, out_ref, scr_ref, sem):
    inp = inp_ref[...]
    t48 = t_ref[pl.ds(PA * P, P), :]
    t49 = t_ref[pl.ds((PA + 1) * P, P), :]
    iota = lax.broadcasted_iota(jnp.int32, (BB, P), 1)
    oh48 = (inp[:, 48:49] == iota).astype(jnp.float32)
    oh49 = (inp[:, 49:50] == iota).astype(jnp.float32)
    scr_ref[:, 0, :] = jnp.dot(oh48, t48, preferred_element_type=jnp.float32,
                               precision=lax.Precision.HIGHEST)
    scr_ref[:, 1, :] = jnp.dot(oh49, t49, preferred_element_type=jnp.float32,
                               precision=lax.Precision.HIGHEST)
    i = pl.program_id(0)
    pltpu.make_async_copy(
        scr_ref, out_ref.at[pl.ds(i * BB, BB), pl.ds(PA, P - PA)], sem
    ).start()
    pltpu.make_async_copy(
        scr_ref, out_ref.at[pl.ds(i * BB, BB), pl.ds(PA, P - PA)], sem
    ).wait()


def kernel(input, embedding, pos_embedding):
    t, idx = pl.pallas_call(
        _build_table_tc,
        out_shape=(
            jax.ShapeDtypeStruct((P, P, D), jnp.float32),
            jax.ShapeDtypeStruct((B, PPAD), jnp.int32),
        ),
    )(input.astype(jnp.int32), embedding, pos_embedding)

    t = t.reshape(P * P, D)
    idx_flat = idx.reshape(B * PPAD)

    sc = functools.partial(
        pl.kernel,
        out_type=jax.ShapeDtypeStruct((B, P, D), jnp.float32),
        mesh=plsc.VectorSubcoreMesh(
            core_axis_name="c", subcore_axis_name="s",
            num_cores=2, num_subcores=16),
        scratch_types=[
            pltpu.VMEM((B_PER_W * PPAD,), jnp.int32),
            tuple(pltpu.VMEM((PPAD, D), jnp.float32) for _ in range(NBUF)),
            tuple(pltpu.SemaphoreType.DMA for _ in range(NBUF)),
            tuple(pltpu.SemaphoreType.DMA for _ in range(NBUF)),
        ],
    )(_sc_gather)

    return sc(t, idx_flat)


# speedup vs baseline: 3.0334x; 3.0334x over previous
import functools

import jax
import jax.numpy as jnp
from jax import lax
from jax.experimental import pallas as pl
from jax.experimental.pallas import tpu as pltpu
from jax.experimental.pallas import tpu_sc as plsc

P = 50
PPAD = 56
PA = 48
D = 512
B = 4096
NW = 32
B_PER_W = B // NW
NBUF = 4
NITER = B_PER_W // NBUF
BB = 256


def _build_table_tc(inp_ref, emb_ref, pos_ref, t_ref, idx_ref):
    emb = emb_ref[...]
    pos = pos_ref[...]
    t_ref[...] = pos[:, None, :] + emb[None, :, :]
    inp2 = jnp.concatenate([inp_ref[...], inp_ref[:, 42:48]], axis=1)
    j = lax.broadcasted_iota(jnp.int32, (B, PPAD), 1)
    p2 = jnp.where(j < P, j, j - 8)
    idx_ref[...] = inp2 + P * p2


def _sc_gather(t_hbm, idx_hbm, out_hbm, idx_v, bufs, gsems, ssems):
    wid = lax.axis_index("s") * 2 + lax.axis_index("c")
    b0 = wid * B_PER_W
    pltpu.sync_copy(idx_hbm.at[pl.ds(b0 * PPAD, B_PER_W * PPAD)], idx_v)

    def gather(bl, slot):
        off = pl.multiple_of(bl * PPAD, PPAD)
        return pltpu.make_async_copy(
            t_hbm.at[idx_v.at[pl.ds(off, PPAD)]], bufs[slot], gsems[slot])

    def store(bl, slot):
        return pltpu.make_async_copy(
            bufs[slot], out_hbm.at[b0 + bl, pl.ds(0, PPAD)], ssems[slot])

    gather(0, 0).start()

    def body(g, carry):
        c0 = NBUF * g
        for j in range(NBUF):
            c = c0 + j
            nslot = (j + 1) % NBUF
            if j < NBUF - 1:
                @pl.when(g > 0)
                def _():
                    store(c + 1 - NBUF, nslot).wait()
                gather(c + 1, nslot).start()
            else:
                @pl.when(g < NITER - 1)
                def _():
                    store(c + 1 - NBUF, nslot).wait()
                    gather(c + 1, nslot).start()
            gather(c, j).wait()
            store(c, j).start()
        return carry

    lax.fori_loop(0, NITER, body, 0)
    for j in range(NBUF):
        store(B_PER_W - NBUF + j, j).wait()


def _fix_tail_tc(inp_ref, t_ref, prev_ref, out_ref, scr_ref, sem):
    inp = inp_ref[...]
    t48 = t_ref[pl.ds(PA * P, P), :]
    t49 = t_ref[pl.ds((PA + 1) * P, P), :]
    iota = lax.broadcasted_iota(jnp.int32, (BB, P), 1)
    oh48 = (inp[:, 48:49] == iota).astype(jnp.float32)
    oh49 = (inp[:, 49:50] == iota).astype(jnp.float32)
    scr_ref[:, 0, :] = jnp.dot(oh48, t48, preferred_element_type=jnp.float32,
                               precision=lax.Precision.HIGHEST)
    scr_ref[:, 1, :] = jnp.dot(oh49, t49, preferred_element_type=jnp.float32,
                               precision=lax.Precision.HIGHEST)
    i = pl.program_id(0)
    pltpu.make_async_copy(
        scr_ref, out_ref.at[pl.ds(i * BB, BB), pl.ds(PA, P - PA)], sem
    ).start()
    pltpu.make_async_copy(
        scr_ref, out_ref.at[pl.ds(i * BB, BB), pl.ds(PA, P - PA)], sem
    ).wait()


def kernel(input, embedding, pos_embedding):
    t, idx = pl.pallas_call(
        _build_table_tc,
        out_shape=(
            jax.ShapeDtypeStruct((P, P, D), jnp.float32),
            jax.ShapeDtypeStruct((B, PPAD), jnp.int32),
        ),
    )(input.astype(jnp.int32), embedding, pos_embedding)

    t = t.reshape(P * P, D)
    idx_flat = idx.reshape(B * PPAD)

    sc = functools.partial(
        pl.kernel,
        out_type=jax.ShapeDtypeStruct((B, P, D), jnp.float32),
        mesh=plsc.VectorSubcoreMesh(
            core_axis_name="c", subcore_axis_name="s",
            num_cores=2, num_subcores=16),
        scratch_types=[
            pltpu.VMEM((B_PER_W * PPAD,), jnp.int32),
            tuple(pltpu.VMEM((PPAD, D), jnp.float32) for _ in range(NBUF)),
            tuple(pltpu.SemaphoreType.DMA for _ in range(NBUF)),
            tuple(pltpu.SemaphoreType.DMA for _ in range(NBUF)),
        ],
    )(_sc_gather)

    return sc(t, idx_flat)


# speedup vs baseline: 3.1045x; 1.0234x over previous
import functools

import jax
import jax.numpy as jnp
from jax import lax
from jax.experimental import pallas as pl
from jax.experimental.pallas import tpu as pltpu
from jax.experimental.pallas import tpu_sc as plsc

P = 50
PPAD = 56
PA = 48
D = 512
B = 4096
NW = 32
B_PER_W = B // NW
NBUF = 4
NITER = B_PER_W // NBUF
BB = 256


def _build_table_tc(inp_ref, emb_ref, pos_ref, t_ref, idx_ref):
    emb = emb_ref[...]
    pos = pos_ref[...]
    t_ref[...] = pos[:, None, :] + emb[None, :, :]
    inp_pad = jnp.concatenate(
        [inp_ref[...], jnp.zeros((B, PPAD - P), jnp.int32)], axis=1)
    j = lax.broadcasted_iota(jnp.int32, (B, PPAD), 1)
    idx_ref[...] = jnp.where(j < P, inp_pad + P * j, 0)


def _sc_gather(t_hbm, idx_hbm, out_hbm, idx_v, bufs, gsems, ssems):
    wid = lax.axis_index("s") * 2 + lax.axis_index("c")
    b0 = wid * B_PER_W
    pltpu.sync_copy(idx_hbm.at[pl.ds(b0 * PPAD, B_PER_W * PPAD)], idx_v)

    def gather(bl, slot):
        off = pl.multiple_of(bl * PPAD, PPAD)
        return pltpu.make_async_copy(
            t_hbm.at[idx_v.at[pl.ds(off, PA)]], bufs[slot], gsems[slot])

    def store(bl, slot):
        return pltpu.make_async_copy(
            bufs[slot], out_hbm.at[b0 + bl, pl.ds(0, PA)], ssems[slot])

    gather(0, 0).start()

    def body(g, carry):
        c0 = NBUF * g
        for j in range(NBUF):
            c = c0 + j
            nslot = (j + 1) % NBUF
            if j < NBUF - 1:
                @pl.when(g > 0)
                def _():
                    store(c + 1 - NBUF, nslot).wait()
                gather(c + 1, nslot).start()
            else:
                @pl.when(g < NITER - 1)
                def _():
                    store(c + 1 - NBUF, nslot).wait()
                    gather(c + 1, nslot).start()
            gather(c, j).wait()
            store(c, j).start()
        return carry

    lax.fori_loop(0, NITER, body, 0)
    for j in range(NBUF):
        store(B_PER_W - NBUF + j, j).wait()


def _fix_tail_tc(inp_ref, t_ref, prev_ref, out_ref, scr_ref, sem):
    inp = inp_ref[...]
    t48 = t_ref[pl.ds(PA * P, P), :]
    t49 = t_ref[pl.ds((PA + 1) * P, P), :]
    iota = lax.broadcasted_iota(jnp.int32, (BB, P), 1)
    oh48 = (inp[:, 48:49] == iota).astype(jnp.float32)
    oh49 = (inp[:, 49:50] == iota).astype(jnp.float32)
    scr_ref[:, 0, :] = jnp.dot(oh48, t48, preferred_element_type=jnp.float32,
                               precision=lax.Precision.HIGHEST)
    scr_ref[:, 1, :] = jnp.dot(oh49, t49, preferred_element_type=jnp.float32,
                               precision=lax.Precision.HIGHEST)
    i = pl.program_id(0)
    pltpu.make_async_copy(
        scr_ref, out_ref.at[pl.ds(i * BB, BB), pl.ds(PA, P - PA)], sem
    ).start()
    pltpu.make_async_copy(
        scr_ref, out_ref.at[pl.ds(i * BB, BB), pl.ds(PA, P - PA)], sem
    ).wait()


def kernel(input, embedding, pos_embedding):
    t, idx = pl.pallas_call(
        _build_table_tc,
        out_shape=(
            jax.ShapeDtypeStruct((P, P, D), jnp.float32),
            jax.ShapeDtypeStruct((B, PPAD), jnp.int32),
        ),
    )(input.astype(jnp.int32), embedding, pos_embedding)

    t = t.reshape(P * P, D)
    idx_flat = idx.reshape(B * PPAD)

    sc = functools.partial(
        pl.kernel,
        out_type=jax.ShapeDtypeStruct((B, P, D), jnp.float32),
        mesh=plsc.VectorSubcoreMesh(
            core_axis_name="c", subcore_axis_name="s",
            num_cores=2, num_subcores=16),
        scratch_types=[
            pltpu.VMEM((B_PER_W * PPAD,), jnp.int32),
            tuple(pltpu.VMEM((PA, D), jnp.float32) for _ in range(NBUF)),
            tuple(pltpu.SemaphoreType.DMA for _ in range(NBUF)),
            tuple(pltpu.SemaphoreType.DMA for _ in range(NBUF)),
        ],
    )(_sc_gather)

    res = sc(t, idx_flat)

    out = pl.pallas_call(
        _fix_tail_tc,
        grid=(B // BB,),
        in_specs=[
            pl.BlockSpec((BB, P), lambda i: (i, 0)),
            pl.BlockSpec((P * P, D), lambda i: (0, 0)),
            pl.BlockSpec(memory_space=pltpu.HBM),
        ],
        out_specs=pl.BlockSpec(memory_space=pltpu.HBM),
        out_shape=jax.ShapeDtypeStruct((B, P, D), jnp.float32),
        scratch_shapes=[
            pltpu.VMEM((BB, P - PA, D), jnp.float32),
            pltpu.SemaphoreType.DMA,
        ],
        input_output_aliases={2: 0},
    )(input.astype(jnp.int32), t, res)
    return out
